# baseline (speedup 1.0000x reference)
import functools

import jax
import jax.numpy as jnp
from jax import lax
from jax.experimental import pallas as pl
from jax.experimental.pallas import tpu as pltpu
from jax.experimental.pallas import tpu_sc as plsc

N = 10000
E = 320000
D_IN = 128
D_H = 256
D_OUT = 64

NC = 2
NS = 16
NW = NC * NS

B = 128
ITERS = 80
NG = ITERS // 2
E_PAD = NW * ITERS * B
RPT = 632
N_PAD = NS * RPT
DUMMY = N

CA = 64
CD = 16

_SC_PARAMS = pltpu.CompilerParams(use_tc_tiling_on_sc=False)


def _sc_agg_body(K, *args):
    xs = args[:K]
    edges, zeros, out = args[K:K + 3]
    e_all, rows0, rows1, aggS, sg0, sg1, ss0, ss1 = args[K + 3:]
    c = lax.axis_index("c")
    s = lax.axis_index("s")
    wid = c * NS + s
    r0 = s * RPT
    pltpu.sync_copy(edges.at[wid], e_all)
    for k in range(K):
        xk = xs[k]
        pltpu.sync_copy(zeros.at[pl.ds(r0, RPT)], aggS.at[pl.ds(r0, RPT)])
        plsc.subcore_barrier()

        def fire_g(j, buf, sem, xk=xk):
            pltpu.async_copy(xk.at[e_all.at[j, 0]], buf, sem)

        def wait_g(j, buf, sem, xk=xk):
            pltpu.make_async_copy(xk.at[e_all.at[j, 0]], buf, sem).wait()

        def fire_s(j, buf, sem):
            pltpu.async_copy(buf, aggS.at[e_all.at[j, 1]], sem, add=True)

        def wait_s(j, buf, sem):
            pltpu.make_async_copy(buf, aggS.at[e_all.at[j, 1]], sem).wait()

        fire_g(0, rows0, sg0)

        def it(i, carry):
            j = 2 * i
            fire_g(j + 1, rows1, sg1)
            wait_g(j, rows0, sg0)
            fire_s(j, rows0, ss0)
            wait_s(j, rows0, ss0)

            @pl.when(i < NG - 1)
            def _():
                fire_g(j + 2, rows0, sg0)

            wait_g(j + 1, rows1, sg1)
            fire_s(j + 1, rows1, ss1)
            wait_s(j + 1, rows1, ss1)
            return carry

        lax.fori_loop(0, NG, it, 0)
        plsc.subcore_barrier()
        pltpu.sync_copy(aggS.at[pl.ds(r0, RPT)], out.at[c, k, pl.ds(r0, RPT)])
        if k + 1 < K:
            plsc.subcore_barrier()


@functools.cache
def _make_sc_agg(K):
    mesh = plsc.VectorSubcoreMesh(core_axis_name="c", subcore_axis_name="s")
    return pl.kernel(
        functools.partial(_sc_agg_body, K),
        out_type=jax.ShapeDtypeStruct((NC, K, N_PAD, CA), jnp.float32),
        mesh=mesh,
        scratch_types=[
            pltpu.VMEM((ITERS, 2, B), jnp.int32),
            pltpu.VMEM((B, CA), jnp.float32),
            pltpu.VMEM((B, CA), jnp.float32),
            pltpu.VMEM_SHARED((N_PAD, CA), jnp.float32),
            pltpu.SemaphoreType.DMA,
            pltpu.SemaphoreType.DMA,
            pltpu.SemaphoreType.DMA,
            pltpu.SemaphoreType.DMA,
        ],
        compiler_params=_SC_PARAMS,
    )


def _sc_agg(xchunks, edges):
    K = len(xchunks)
    zeros = jnp.zeros((N_PAD, CA), jnp.float32)
    return _make_sc_agg(K)(*xchunks, edges, zeros)


def _sc_deg_body(edges, ones_h, zeros_d, out_d, e_all, ones_v, degS, ss0, ss1):
    c = lax.axis_index("c")
    s = lax.axis_index("s")
    wid = c * NS + s
    r0 = s * RPT
    pltpu.sync_copy(edges.at[wid], e_all)
    pltpu.sync_copy(ones_h, ones_v)
    pltpu.sync_copy(zeros_d.at[pl.ds(r0, RPT)], degS.at[pl.ds(r0, RPT)])
    plsc.subcore_barrier()

    def fire_s(j, sem):
        pltpu.async_copy(ones_v, degS.at[e_all.at[j, 1]], sem, add=True)

    def wait_s(j, sem):
        pltpu.make_async_copy(ones_v, degS.at[e_all.at[j, 1]], sem).wait()

    def it(i, carry):
        j = 2 * i
        fire_s(j, ss0)
        fire_s(j + 1, ss1)
        wait_s(j, ss0)
        wait_s(j + 1, ss1)
        return carry

    lax.fori_loop(0, NG, it, 0)
    plsc.subcore_barrier()
    pltpu.sync_copy(degS.at[pl.ds(r0, RPT)], out_d.at[c, pl.ds(r0, RPT)])


@functools.cache
def _make_sc_deg():
    mesh = plsc.VectorSubcoreMesh(core_axis_name="c", subcore_axis_name="s")
    return pl.kernel(
        _sc_deg_body,
        out_type=jax.ShapeDtypeStruct((NC, N_PAD, CD), jnp.float32),
        mesh=mesh,
        scratch_types=[
            pltpu.VMEM((ITERS, 2, B), jnp.int32),
            pltpu.VMEM((B, CD), jnp.float32),
            pltpu.VMEM_SHARED((N_PAD, CD), jnp.float32),
            pltpu.SemaphoreType.DMA,
            pltpu.SemaphoreType.DMA,
        ],
        compiler_params=_SC_PARAMS,
    )


def _deg_of(pd_ref):
    return jnp.maximum(pd_ref[0][:, 0:1] + pd_ref[1][:, 0:1], 1.0)


def _mm0_body(x_ref, p_ref, pd_ref, ws_ref, wn_ref, b_ref, h_ref, hc_ref):
    hn = jnp.concatenate(
        [p_ref[0, 0] + p_ref[1, 0], p_ref[0, 1] + p_ref[1, 1]],
        axis=1) / _deg_of(pd_ref)
    h = (jnp.dot(x_ref[...], ws_ref[...], preferred_element_type=jnp.float32)
         + jnp.dot(hn, wn_ref[...], preferred_element_type=jnp.float32)
         + b_ref[...])
    h = jnp.maximum(h, 0.0)
    h_ref[...] = h
    for j in range(4):
        hc_ref[j] = h[:, CA * j:CA * (j + 1)]


def _mm1_body(h0_ref, p1_ref, pd_ref, ws_ref, wn_ref, b_ref, wn2_ref,
              h1_ref, y2_ref):
    hn = jnp.concatenate(
        [p1_ref[0, j] + p1_ref[1, j] for j in range(4)],
        axis=1) / _deg_of(pd_ref)
    h1 = (jnp.dot(h0_ref[...], ws_ref[...], preferred_element_type=jnp.float32)
          + jnp.dot(hn, wn_ref[...], preferred_element_type=jnp.float32)
          + b_ref[...])
    h1 = jnp.maximum(h1, 0.0)
    h1_ref[...] = h1
    y2_ref[...] = jnp.dot(h1, wn2_ref[...], preferred_element_type=jnp.float32)


def _mm2_body(h1_ref, p2_ref, pd_ref, ws_ref, b_ref, o_ref):
    hn = (p2_ref[0] + p2_ref[1]) / _deg_of(pd_ref)
    o_ref[...] = (jnp.dot(h1_ref[...], ws_ref[...],
                          preferred_element_type=jnp.float32)
                  + hn + b_ref[...])


BN = 1000
_G = N // BN


def _full(shape):
    return pl.BlockSpec(shape, lambda i: tuple(0 for _ in shape))


def _rows(shape):
    nd = len(shape)
    return pl.BlockSpec(shape, lambda i, nd=nd: tuple(
        i if d == nd - 2 else 0 for d in range(nd)))


def kernel(inputs, edge_index, W_self0, W_neigh0, b0, W_self1, W_neigh1, b1,
           W_self2, W_neigh2, b2):
    x = inputs
    pad = E_PAD - E
    src = jnp.concatenate([edge_index[0], jnp.zeros((pad,), jnp.int32)])
    dst = jnp.concatenate([edge_index[1], jnp.full((pad,), DUMMY, jnp.int32)])
    edges = (jnp.stack([src, dst])
             .reshape(2, NW, ITERS, B).transpose(1, 2, 0, 3))

    pd = _make_sc_deg()(edges, jnp.ones((B, CD), jnp.float32),
                        jnp.zeros((N_PAD, CD), jnp.float32))

    x0 = lax.slice(x, (0, 0), (N, CA))
    x1 = lax.slice(x, (0, CA), (N, 2 * CA))
    p0 = _sc_agg([x0, x1], edges)

    h0, h0c = pl.pallas_call(
        _mm0_body,
        grid=(_G,),
        in_specs=[
            _rows((BN, D_IN)),
            _rows((2, 2, BN, CA)),
            _rows((2, BN, CD)),
            _full((D_IN, D_H)),
            _full((D_IN, D_H)),
            _full((1, D_H)),
        ],
        out_specs=[_rows((BN, D_H)), _rows((4, BN, CA))],
        out_shape=[jax.ShapeDtypeStruct((N, D_H), jnp.float32),
                   jax.ShapeDtypeStruct((4, N, CA), jnp.float32)],
    )(x, p0, pd, W_self0, W_neigh0, b0.reshape(1, -1))

    p1 = _sc_agg([h0c[0], h0c[1], h0c[2], h0c[3]], edges)

    h1, y2 = pl.pallas_call(
        _mm1_body,
        grid=(_G,),
        in_specs=[
            _rows((BN, D_H)),
            _rows((2, 4, BN, CA)),
            _rows((2, BN, CD)),
            _full((D_H, D_H)),
            _full((D_H, D_H)),
            _full((1, D_H)),
            _full((D_H, D_OUT)),
        ],
        out_specs=[_rows((BN, D_H)), _rows((BN, D_OUT))],
        out_shape=[jax.ShapeDtypeStruct((N, D_H), jnp.float32),
                   jax.ShapeDtypeStruct((N, D_OUT), jnp.float32)],
    )(h0, p1, pd, W_self1, W_neigh1, b1.reshape(1, -1), W_neigh2)

    p2 = _sc_agg([y2], edges)[:, 0]

    out = pl.pallas_call(
        _mm2_body,
        grid=(_G,),
        in_specs=[
            _rows((BN, D_H)),
            _rows((2, BN, CA)),
            _rows((2, BN, CD)),
            _full((D_H, D_OUT)),
            _full((1, D_OUT)),
        ],
        out_specs=_rows((BN, D_OUT)),
        out_shape=jax.ShapeDtypeStruct((N, D_OUT), jnp.float32),
    )(h1, p2, pd, W_self2, b2.reshape(1, -1))

    return (out, h0, h1)


# speedup vs baseline: 2.3934x; 2.3934x over previous
import functools

import jax
import jax.numpy as jnp
from jax import lax
from jax.experimental import pallas as pl
from jax.experimental.pallas import tpu as pltpu
from jax.experimental.pallas import tpu_sc as plsc

N = 10000
E = 320000
D_IN = 128
D_H = 256
D_OUT = 64

NC = 2
NS = 16
NW = NC * NS

B = 128
ITERS = 80
NG = ITERS // 2
E_PAD = NW * ITERS * B
RPT = 632
N_PAD = NS * RPT
DUMMY = N

CA = 64
CD = 16

_SC_PARAMS = pltpu.CompilerParams(use_tc_tiling_on_sc=False)


def _sc_agg_body(K, *args):
    xs = args[:K]
    edges, zeros, out = args[K:K + 3]
    e_all, rows0, rows1, aggS, xS, sg0, sg1, ss0, ss1 = args[K + 3:]
    c = lax.axis_index("c")
    s = lax.axis_index("s")
    wid = c * NS + s
    r0 = s * RPT
    pltpu.sync_copy(edges.at[wid], e_all)
    for k in range(K):
        xk = xs[k]
        pltpu.sync_copy(zeros.at[pl.ds(r0, RPT)], aggS.at[pl.ds(r0, RPT)])
        pltpu.sync_copy(xk.at[pl.ds(r0, RPT)], xS.at[pl.ds(r0, RPT)])
        plsc.subcore_barrier()

        def fire_g(j, buf, sem):
            pltpu.async_copy(xS.at[e_all.at[j, 0]], buf, sem)

        def wait_g(j, buf, sem):
            pltpu.make_async_copy(xS.at[e_all.at[j, 0]], buf, sem).wait()

        def fire_s(j, buf, sem):
            pltpu.async_copy(buf, aggS.at[e_all.at[j, 1]], sem, add=True)

        def wait_s(j, buf, sem):
            pltpu.make_async_copy(buf, aggS.at[e_all.at[j, 1]], sem).wait()

        fire_g(0, rows0, sg0)

        def it(i, carry):
            j = 2 * i
            fire_g(j + 1, rows1, sg1)
            wait_g(j, rows0, sg0)
            fire_s(j, rows0, ss0)
            wait_s(j, rows0, ss0)

            @pl.when(i < NG - 1)
            def _():
                fire_g(j + 2, rows0, sg0)

            wait_g(j + 1, rows1, sg1)
            fire_s(j + 1, rows1, ss1)
            wait_s(j + 1, rows1, ss1)
            return carry

        lax.fori_loop(0, NG, it, 0)
        plsc.subcore_barrier()
        pltpu.sync_copy(aggS.at[pl.ds(r0, RPT)], out.at[c, k, pl.ds(r0, RPT)])
        if k + 1 < K:
            plsc.subcore_barrier()


@functools.cache
def _make_sc_agg(K):
    mesh = plsc.VectorSubcoreMesh(core_axis_name="c", subcore_axis_name="s")
    return pl.kernel(
        functools.partial(_sc_agg_body, K),
        out_type=jax.ShapeDtypeStruct((NC, K, N_PAD, CA), jnp.float32),
        mesh=mesh,
        scratch_types=[
            pltpu.VMEM((ITERS, 2, B), jnp.int32),
            pltpu.VMEM((B, CA), jnp.float32),
            pltpu.VMEM((B, CA), jnp.float32),
            pltpu.VMEM_SHARED((N_PAD, CA), jnp.float32),
            pltpu.VMEM_SHARED((N_PAD, CA), jnp.float32),
            pltpu.SemaphoreType.DMA,
            pltpu.SemaphoreType.DMA,
            pltpu.SemaphoreType.DMA,
            pltpu.SemaphoreType.DMA,
        ],
        compiler_params=_SC_PARAMS,
    )


def _sc_agg(xchunks, edges):
    K = len(xchunks)
    zeros = jnp.zeros((N_PAD, CA), jnp.float32)
    return _make_sc_agg(K)(*xchunks, edges, zeros)


def _sc_deg_body(edges, ones_h, zeros_d, out_d, e_all, ones_v, degS, ss0, ss1):
    c = lax.axis_index("c")
    s = lax.axis_index("s")
    wid = c * NS + s
    r0 = s * RPT
    pltpu.sync_copy(edges.at[wid], e_all)
    pltpu.sync_copy(ones_h, ones_v)
    pltpu.sync_copy(zeros_d.at[pl.ds(r0, RPT)], degS.at[pl.ds(r0, RPT)])
    plsc.subcore_barrier()

    def fire_s(j, sem):
        pltpu.async_copy(ones_v, degS.at[e_all.at[j, 1]], sem, add=True)

    def wait_s(j, sem):
        pltpu.make_async_copy(ones_v, degS.at[e_all.at[j, 1]], sem).wait()

    def it(i, carry):
        j = 2 * i
        fire_s(j, ss0)
        fire_s(j + 1, ss1)
        wait_s(j, ss0)
        wait_s(j + 1, ss1)
        return carry

    lax.fori_loop(0, NG, it, 0)
    plsc.subcore_barrier()
    pltpu.sync_copy(degS.at[pl.ds(r0, RPT)], out_d.at[c, pl.ds(r0, RPT)])


@functools.cache
def _make_sc_deg():
    mesh = plsc.VectorSubcoreMesh(core_axis_name="c", subcore_axis_name="s")
    return pl.kernel(
        _sc_deg_body,
        out_type=jax.ShapeDtypeStruct((NC, N_PAD, CD), jnp.float32),
        mesh=mesh,
        scratch_types=[
            pltpu.VMEM((ITERS, 2, B), jnp.int32),
            pltpu.VMEM((B, CD), jnp.float32),
            pltpu.VMEM_SHARED((N_PAD, CD), jnp.float32),
            pltpu.SemaphoreType.DMA,
            pltpu.SemaphoreType.DMA,
        ],
        compiler_params=_SC_PARAMS,
    )


def _deg_of(pd_ref):
    return jnp.maximum(pd_ref[0][:, 0:1] + pd_ref[1][:, 0:1], 1.0)


def _mm0_body(x_ref, p_ref, pd_ref, ws_ref, wn_ref, b_ref, h_ref, hc_ref):
    hn = jnp.concatenate(
        [p_ref[0, 0] + p_ref[1, 0], p_ref[0, 1] + p_ref[1, 1]],
        axis=1) / _deg_of(pd_ref)
    h = (jnp.dot(x_ref[...], ws_ref[...], preferred_element_type=jnp.float32)
         + jnp.dot(hn, wn_ref[...], preferred_element_type=jnp.float32)
         + b_ref[...])
    h = jnp.maximum(h, 0.0)
    h_ref[...] = h
    for j in range(4):
        hc_ref[j] = h[:, CA * j:CA * (j + 1)]


def _mm1_body(h0_ref, p1_ref, pd_ref, ws_ref, wn_ref, b_ref, wn2_ref,
              h1_ref, y2_ref):
    hn = jnp.concatenate(
        [p1_ref[0, j] + p1_ref[1, j] for j in range(4)],
        axis=1) / _deg_of(pd_ref)
    h1 = (jnp.dot(h0_ref[...], ws_ref[...], preferred_element_type=jnp.float32)
          + jnp.dot(hn, wn_ref[...], preferred_element_type=jnp.float32)
          + b_ref[...])
    h1 = jnp.maximum(h1, 0.0)
    h1_ref[...] = h1
    y2_ref[...] = jnp.dot(h1, wn2_ref[...], preferred_element_type=jnp.float32)


def _mm2_body(h1_ref, p2_ref, pd_ref, ws_ref, b_ref, o_ref):
    hn = (p2_ref[0] + p2_ref[1]) / _deg_of(pd_ref)
    o_ref[...] = (jnp.dot(h1_ref[...], ws_ref[...],
                          preferred_element_type=jnp.float32)
                  + hn + b_ref[...])


BN = 1000
_G = N // BN


def _full(shape):
    return pl.BlockSpec(shape, lambda i: tuple(0 for _ in shape))


def _rows(shape):
    nd = len(shape)
    return pl.BlockSpec(shape, lambda i, nd=nd: tuple(
        i if d == nd - 2 else 0 for d in range(nd)))


def kernel(inputs, edge_index, W_self0, W_neigh0, b0, W_self1, W_neigh1, b1,
           W_self2, W_neigh2, b2):
    x = inputs
    pad = E_PAD - E
    src = jnp.concatenate([edge_index[0], jnp.zeros((pad,), jnp.int32)])
    dst = jnp.concatenate([edge_index[1], jnp.full((pad,), DUMMY, jnp.int32)])
    edges = (jnp.stack([src, dst])
             .reshape(2, NW, ITERS, B).transpose(1, 2, 0, 3))

    pd = _make_sc_deg()(edges, jnp.ones((B, CD), jnp.float32),
                        jnp.zeros((N_PAD, CD), jnp.float32))

    x_pad = jnp.pad(x, ((0, N_PAD - N), (0, 0)))
    x0 = lax.slice(x_pad, (0, 0), (N_PAD, CA))
    x1 = lax.slice(x_pad, (0, CA), (N_PAD, 2 * CA))
    p0 = _sc_agg([x0, x1], edges)

    h0, h0c = pl.pallas_call(
        _mm0_body,
        grid=(_G,),
        in_specs=[
            _rows((BN, D_IN)),
            _rows((2, 2, BN, CA)),
            _rows((2, BN, CD)),
            _full((D_IN, D_H)),
            _full((D_IN, D_H)),
            _full((1, D_H)),
        ],
        out_specs=[_rows((BN, D_H)), _rows((4, BN, CA))],
        out_shape=[jax.ShapeDtypeStruct((N, D_H), jnp.float32),
                   jax.ShapeDtypeStruct((4, N_PAD, CA), jnp.float32)],
    )(x, p0, pd, W_self0, W_neigh0, b0.reshape(1, -1))

    p1 = _sc_agg([h0c[0], h0c[1], h0c[2], h0c[3]], edges)

    h1, y2 = pl.pallas_call(
        _mm1_body,
        grid=(_G,),
        in_specs=[
            _rows((BN, D_H)),
            _rows((2, 4, BN, CA)),
            _rows((2, BN, CD)),
            _full((D_H, D_H)),
            _full((D_H, D_H)),
            _full((1, D_H)),
            _full((D_H, D_OUT)),
        ],
        out_specs=[_rows((BN, D_H)), _rows((BN, D_OUT))],
        out_shape=[jax.ShapeDtypeStruct((N, D_H), jnp.float32),
                   jax.ShapeDtypeStruct((N_PAD, D_OUT), jnp.float32)],
    )(h0, p1, pd, W_self1, W_neigh1, b1.reshape(1, -1), W_neigh2)

    p2 = _sc_agg([y2], edges)[:, 0]

    out = pl.pallas_call(
        _mm2_body,
        grid=(_G,),
        in_specs=[
            _rows((BN, D_H)),
            _rows((2, BN, CA)),
            _rows((2, BN, CD)),
            _full((D_H, D_OUT)),
            _full((1, D_OUT)),
        ],
        out_specs=_rows((BN, D_OUT)),
        out_shape=jax.ShapeDtypeStruct((N, D_OUT), jnp.float32),
    )(h1, p2, pd, W_self2, b2.reshape(1, -1))

    return (out, h0, h1)
